# Initial kernel scaffold; baseline (speedup 1.0000x reference)
import jax
import jax.numpy as jnp
from jax.experimental import pallas as pl


def kernel(hidden_states, gate_w, w1, w3, w2, prefetch_expert_idx):
    raise NotImplementedError("write your pallas kernel here")


# speedup vs baseline: 1.5952x; 1.5952x over previous
import jax
import jax.numpy as jnp
from jax.experimental import pallas as pl
from jax.experimental.pallas import tpu as pltpu

E = 8
TOP_K = 2
H = 1024
FF = 3584
FF_T = 896
N_T = FF // FF_T


def _moe_step(x_ref, gate_ref, w1_ref, w3_ref, w2_ref, out_ref, coef_ref):
    e = pl.program_id(0)
    j = pl.program_id(1)

    @pl.when((e == 0) & (j == 0))
    def _router_and_init():
        x = x_ref[...]
        logits = jnp.dot(
            x.astype(jnp.bfloat16),
            gate_ref[...].astype(jnp.bfloat16).T,
            preferred_element_type=jnp.float32,
        )
        m = jnp.max(logits, axis=1, keepdims=True)
        p = jnp.exp(logits - m)
        p = p / jnp.sum(p, axis=1, keepdims=True)
        idx = jax.lax.broadcasted_iota(jnp.int32, p.shape, 1)
        v1 = jnp.max(p, axis=1, keepdims=True)
        i1 = jnp.min(jnp.where(p == v1, idx, E), axis=1, keepdims=True)
        p2 = jnp.where(idx == i1, -jnp.inf, p)
        v2 = jnp.max(p2, axis=1, keepdims=True)
        i2 = jnp.min(jnp.where(p2 == v2, idx, E), axis=1, keepdims=True)
        sel = jnp.where(idx == i1, v1, 0.0) + jnp.where(idx == i2, v2, 0.0)
        coef_ref[...] = sel / (v1 + v2)
        out_ref[...] = jnp.zeros_like(out_ref)

    xb = x_ref[...].astype(jnp.bfloat16)
    w1b = w1_ref[0].astype(jnp.bfloat16)
    w3b = w3_ref[0].astype(jnp.bfloat16)
    a = jnp.dot(xb, w1b.T, preferred_element_type=jnp.float32)
    b = jnp.dot(xb, w3b.T, preferred_element_type=jnp.float32)
    t = (a * jax.nn.sigmoid(a)) * b

    idx = jax.lax.broadcasted_iota(jnp.int32, coef_ref.shape, 1)
    coef_col = jnp.sum(
        jnp.where(idx == e, coef_ref[...], 0.0), axis=1, keepdims=True
    )
    t = t * coef_col

    w2b = w2_ref[0].astype(jnp.bfloat16)
    out_ref[...] += jnp.dot(
        t.astype(jnp.bfloat16), w2b.T, preferred_element_type=jnp.float32
    )


def kernel(hidden_states, gate_w, w1, w3, w2, prefetch_expert_idx):
    b, s, h = hidden_states.shape
    x = hidden_states.reshape(-1, h)
    n = x.shape[0]

    out = pl.pallas_call(
        _moe_step,
        grid=(E, N_T),
        in_specs=[
            pl.BlockSpec((n, H), lambda e, j: (0, 0)),
            pl.BlockSpec((E, H), lambda e, j: (0, 0)),
            pl.BlockSpec((1, FF_T, H), lambda e, j: (e, j, 0)),
            pl.BlockSpec((1, FF_T, H), lambda e, j: (e, j, 0)),
            pl.BlockSpec((1, H, FF_T), lambda e, j: (e, 0, j)),
        ],
        out_specs=pl.BlockSpec((n, H), lambda e, j: (0, 0)),
        out_shape=jax.ShapeDtypeStruct((n, H), jnp.float32),
        scratch_shapes=[pltpu.VMEM((n, E), jnp.float32)],
    )(x, gate_w, w1, w3, w2)
    return out.reshape(b, s, h)
